# baseline (speedup 1.0000x reference)
import functools

import jax
import jax.numpy as jnp
from jax import lax
from jax.experimental import pallas as pl
from jax.experimental.pallas import tpu as pltpu
from jax.experimental.pallas import tpu_sc as plsc

NC = 2
NS = 16
NW = NC * NS

CH = 80
NB = 4
NI = 8
EPT = 10000
ZCH = 640
NPAD = NS * ZCH

_mesh = plsc.VectorSubcoreMesh(core_axis_name="c", subcore_axis_name="s")


def _deg_body(nch, dst_hbm, deg_out, degacc, dib, zeros_v, ones_v, isem, ssem):
    c = lax.axis_index("c")
    s = lax.axis_index("s")
    wid = c * NS + s
    base = wid * EPT

    def _z(i, _):
        zeros_v[pl.ds(i * 16, 16)] = jnp.zeros((16,), jnp.float32)
        return 0
    lax.fori_loop(0, ZCH // 16, _z, 0)

    def _o(i, _):
        ones_v[pl.ds(i * 16, 16)] = jnp.ones((16,), jnp.float32)
        return 0
    lax.fori_loop(0, CH // 16, _o, 0)

    pltpu.sync_copy(zeros_v, degacc.at[pl.ds(s * ZCH, ZCH)])
    plsc.subcore_barrier()

    def _ld_idx(g, b):
        pltpu.async_copy(dst_hbm.at[pl.ds(base + g * CH, CH)], dib[b], isem[b])

    def _wait_idx(b):
        pltpu.make_async_copy(dst_hbm.at[pl.ds(base, CH)], dib[b], isem[b]).wait()

    def _scat(b, u):
        pltpu.async_copy(ones_v, degacc.at[dib[b]], ssem[u], add=True)

    def _wait_scat(u):
        pltpu.make_async_copy(ones_v, degacc.at[dib[0]], ssem[u]).wait()

    for g in range(NB):
        _ld_idx(g, g)
    for g in range(NB):
        _ld_idx(g + NB, (g + NB) % NI)
        _wait_idx(g)
        _scat(g, g)

    def _step(g, v):
        u = v % NB
        _wait_scat(u)
        _ld_idx(g + NB, (v + NB) % NI)
        _wait_idx(v % NI)
        _scat(v % NI, u)

    def _body(b, _):
        for v in range(NI):
            _step(NI * b + NB + v, NB + v)
        return 0
    nfull = (nch - 2 * NB) // NI * NI
    lax.fori_loop(0, nfull // NI, _body, 0)

    for g in range(NB + nfull, nch - NB):
        _step(g, g % NI)
    for g in range(nch - NB, nch):
        _wait_scat(g % NB)
        _wait_idx(g % NI)
        _scat(g % NI, g % NB)

    for u in range(NB):
        _wait_scat(u)

    plsc.subcore_barrier()
    pltpu.sync_copy(degacc.at[pl.ds(s * ZCH, ZCH)],
                    deg_out.at[c, 0, pl.ds(s * ZCH, ZCH)])


def _prop_body(N, D, nch, hs_hbm, src_hbm, dst_hbm, out_hbm, acc,
               sib, dib, rows, isem, gsem, ssem):
    c = lax.axis_index("c")
    s = lax.axis_index("s")
    wid = c * NS + s
    kd = D // 16
    base = wid * EPT
    r0 = rows[0]

    stripe0 = s * ((N // NS) // 8 * 8)
    slen = N - (NS - 1) * ((N // NS) // 8 * 8)

    def _z(r, _):
        for k in range(kd):
            r0[r, pl.ds(k * 16, 16)] = jnp.zeros((16,), jnp.float32)
        return 0
    lax.fori_loop(0, CH, _z, 0)

    def _zc(j, _):
        pltpu.sync_copy(r0.at[pl.ds(0, 40)], acc.at[pl.ds(stripe0 + j * 40, 40)])
        return 0
    lax.fori_loop(0, slen // 40, _zc, 0)
    plsc.subcore_barrier()

    def _ld_idx(g, b):
        pltpu.async_copy(src_hbm.at[pl.ds(base + g * CH, CH)], sib[b], isem[b])
        pltpu.async_copy(dst_hbm.at[pl.ds(base + g * CH, CH)], dib[b], isem[b])

    def _wait_idx(b):
        pltpu.make_async_copy(src_hbm.at[pl.ds(base, CH)], sib[b], isem[b]).wait()
        pltpu.make_async_copy(dst_hbm.at[pl.ds(base, CH)], dib[b], isem[b]).wait()

    def _gather(b, u):
        pltpu.async_copy(hs_hbm.at[sib[b]], rows[u], gsem[u])

    def _wait_gather(u):
        pltpu.make_async_copy(hs_hbm.at[sib[0]], rows[u], gsem[u]).wait()

    def _scat(b, u):
        pltpu.async_copy(rows[u], acc.at[dib[b]], ssem[u], add=True)

    def _wait_scat(u):
        pltpu.make_async_copy(rows[u], acc.at[dib[0]], ssem[u]).wait()

    for g0 in range(NI - 1):
        _ld_idx(g0, g0)
    for g0 in range(NB - 1):
        _wait_idx(g0)
        _gather(g0, g0)
    _wait_idx(3)
    _gather(3, 3)
    _ld_idx(NI - 1, NI - 1)
    _wait_gather(0)
    _scat(0, 0)

    def _step(g, v):
        u = (v + 0) % NB
        w = (v + NB - 1) % NB
        b3 = (v + NB - 1) % NI
        b7 = (v + NI - 1) % NI
        _wait_scat(w)
        _wait_idx(b3)
        _gather(b3, w)
        _ld_idx(g + NI - 1, b7)
        _wait_gather(u)
        _scat(v % NI, u)

    nfull = (nch - NI) // NI * NI

    def _body(bo, _):
        for v in range(NI):
            _step(NI * bo + 1 + v, 1 + v)
        return 0
    lax.fori_loop(0, nfull // NI, _body, 0)

    for g in range(nfull + 1, nch - NI + 1):
        _step(g, g % NI)
    for g in range(nch - NI + 1, nch - NB + 1):
        v = g % NI
        u = g % NB
        w = (g + NB - 1) % NB
        _wait_scat(w)
        _wait_idx((g + NB - 1) % NI)
        _gather((g + NB - 1) % NI, w)
        _wait_gather(u)
        _scat(v, u)
    for g in range(nch - NB + 1, nch):
        u = g % NB
        _wait_gather(u)
        _scat(g % NI, u)

    for u in range(NB):
        _wait_scat(u)

    plsc.subcore_barrier()
    pltpu.sync_copy(acc.at[pl.ds(stripe0, slen)],
                    out_hbm.at[c, pl.ds(stripe0, slen)])


def _deg_call(dstp, nch):
    return pl.kernel(
        functools.partial(_deg_body, nch),
        out_type=jax.ShapeDtypeStruct((NC, 1, NPAD), jnp.float32),
        mesh=_mesh,
        scratch_types=[
            pltpu.VMEM_SHARED((NPAD,), jnp.float32),
            tuple(pltpu.VMEM((CH,), jnp.int32) for _ in range(NI)),
            pltpu.VMEM((ZCH,), jnp.float32),
            pltpu.VMEM((CH,), jnp.float32),
            tuple(pltpu.SemaphoreType.DMA for _ in range(NI)),
            tuple(pltpu.SemaphoreType.DMA for _ in range(NB)),
        ],
    )(dstp)


def _prop_call(hs, srcp, dstp, N, D, nch):
    return pl.kernel(
        functools.partial(_prop_body, N, D, nch),
        out_type=jax.ShapeDtypeStruct((NC, N, D), jnp.float32),
        mesh=_mesh,
        scratch_types=[
            pltpu.VMEM_SHARED((N, D), jnp.float32),
            tuple(pltpu.VMEM((CH,), jnp.int32) for _ in range(NI)),
            tuple(pltpu.VMEM((CH,), jnp.int32) for _ in range(NI)),
            tuple(pltpu.VMEM((CH, D), jnp.float32) for _ in range(NB)),
            tuple(pltpu.SemaphoreType.DMA for _ in range(NI)),
            tuple(pltpu.SemaphoreType.DMA for _ in range(NB)),
            tuple(pltpu.SemaphoreType.DMA for _ in range(NB)),
        ],
    )(hs, srcp, dstp)


BLK = 2000


def _mm_scale_body(x_ref, w_ref, deg_ref, hs_ref):
    dinv = lax.rsqrt(deg_ref[:, 0:1] + deg_ref[:, 1:2] + 1.0)
    hs_ref[...] = jnp.dot(x_ref[...], w_ref[...],
                          preferred_element_type=jnp.float32) * dinv


def _mid_body(a_ref, hs_ref, deg_ref, b_ref, w_ref, out_ref):
    dinv = lax.rsqrt(deg_ref[:, 0:1] + deg_ref[:, 1:2] + 1.0)
    h = (a_ref[0] + a_ref[1] + hs_ref[...]) * dinv + b_ref[...]
    h = jnp.maximum(h, 0.0)
    out_ref[...] = jnp.dot(h, w_ref[...],
                           preferred_element_type=jnp.float32) * dinv


def _final_body(a_ref, hs_ref, deg_ref, b_ref, out_ref):
    dinv = lax.rsqrt(deg_ref[:, 0:1] + deg_ref[:, 1:2] + 1.0)
    out_ref[...] = (a_ref[0] + a_ref[1] + hs_ref[...]) * dinv + b_ref[...]


def _row_spec(D):
    return pl.BlockSpec((BLK, D), lambda i: (i, 0))


def _mm_scale(x, W, degT, N, D):
    return pl.pallas_call(
        _mm_scale_body,
        grid=(N // BLK,),
        in_specs=[
            _row_spec(D),
            pl.BlockSpec((D, D), lambda i: (0, 0)),
            pl.BlockSpec((BLK, 2), lambda i: (i, 0)),
        ],
        out_specs=_row_spec(D),
        out_shape=jax.ShapeDtypeStruct((N, D), jnp.float32),
    )(x, W, degT)


def _acc_spec(D):
    return pl.BlockSpec((NC, BLK, D), lambda i: (0, i, 0))


def _mid(acc, hs, degT, b, W, N, D):
    return pl.pallas_call(
        _mid_body,
        grid=(N // BLK,),
        in_specs=[
            _acc_spec(D), _row_spec(D),
            pl.BlockSpec((BLK, 2), lambda i: (i, 0)),
            pl.BlockSpec((1, D), lambda i: (0, 0)),
            pl.BlockSpec((D, D), lambda i: (0, 0)),
        ],
        out_specs=_row_spec(D),
        out_shape=jax.ShapeDtypeStruct((N, D), jnp.float32),
    )(acc, hs, degT, b, W)


def _final(acc, hs, degT, b, N, D):
    return pl.pallas_call(
        _final_body,
        grid=(N // BLK,),
        in_specs=[
            _acc_spec(D), _row_spec(D),
            pl.BlockSpec((BLK, 2), lambda i: (i, 0)),
            pl.BlockSpec((1, D), lambda i: (0, 0)),
        ],
        out_specs=_row_spec(D),
        out_shape=jax.ShapeDtypeStruct((N, D), jnp.float32),
    )(acc, hs, degT, b)


def kernel(x, edge_index, cache_name, W1, b1, W2, b2):
    N, D = x.shape
    E = edge_index.shape[1]
    nch = EPT // CH
    srcp = edge_index[0]
    dstp = edge_index[1]

    deg_parts = _deg_call(dstp, nch)
    degT = deg_parts[:, 0, :].T[:N]

    hs1 = _mm_scale(x, W1, degT, N, D)
    acc1 = _prop_call(hs1, srcp, dstp, N, D, nch)
    hs2 = _mid(acc1, hs1, degT, b1.reshape(1, D), W2, N, D)
    acc2 = _prop_call(hs2, srcp, dstp, N, D, nch)
    return _final(acc2, hs2, degT, b2.reshape(1, D), N, D)


# speedup vs baseline: 1.0215x; 1.0215x over previous
import functools

import jax
import jax.numpy as jnp
from jax import lax
from jax.experimental import pallas as pl
from jax.experimental.pallas import tpu as pltpu
from jax.experimental.pallas import tpu_sc as plsc

NC = 2
NS = 16
NW = NC * NS

CH = 80
NB = 4
NI = 8
EPT = 10000
ZCH = 640
NPAD = NS * ZCH

_mesh = plsc.VectorSubcoreMesh(core_axis_name="c", subcore_axis_name="s")


def _deg_body(nch, dst_hbm, deg_out, degacc, dstb, zeros_v, ones_v, *sems):
    c = lax.axis_index("c")
    s = lax.axis_index("s")
    wid = c * NS + s

    pltpu.sync_copy(dst_hbm.at[wid], dstb)

    def _z(i, _):
        zeros_v[pl.ds(i * 16, 16)] = jnp.zeros((16,), jnp.float32)
        return 0
    lax.fori_loop(0, ZCH // 16, _z, 0)

    def _o(i, _):
        ones_v[pl.ds(i * 16, 16)] = jnp.ones((16,), jnp.float32)
        return 0
    lax.fori_loop(0, CH // 16, _o, 0)

    pltpu.sync_copy(zeros_v, degacc.at[pl.ds(s * ZCH, ZCH)])
    plsc.subcore_barrier()

    def _issue(g, u):
        pltpu.async_copy(ones_v, degacc.at[dstb.at[g]], sems[u], add=True)

    def _wait(u):
        pltpu.make_async_copy(ones_v, degacc.at[dstb.at[0]], sems[u]).wait()

    for u in range(NB):
        _issue(u, u)

    def _body(b, _):
        for u in range(NB):
            _wait(u)
            _issue(NB * b + u, u)
        return 0
    lax.fori_loop(1, nch // NB, _body, 0)

    for g in range((nch // NB) * NB, nch):
        _wait(g % NB)
        _issue(g, g % NB)

    for u in range(NB):
        _wait(u)

    plsc.subcore_barrier()
    pltpu.sync_copy(degacc.at[pl.ds(s * ZCH, ZCH)],
                    deg_out.at[c, 0, pl.ds(s * ZCH, ZCH)])


def _prop_body(N, D, nch, hs_hbm, src_hbm, dst_hbm, out_hbm, acc,
               sib, dib, rows, isem, gsem, ssem):
    c = lax.axis_index("c")
    s = lax.axis_index("s")
    wid = c * NS + s
    kd = D // 16
    base = wid * EPT
    r0 = rows[0]

    stripe0 = s * ((N // NS) // 8 * 8)
    slen = N - (NS - 1) * ((N // NS) // 8 * 8)

    def _z(r, _):
        for k in range(kd):
            r0[r, pl.ds(k * 16, 16)] = jnp.zeros((16,), jnp.float32)
        return 0
    lax.fori_loop(0, CH, _z, 0)

    def _zc(j, _):
        pltpu.sync_copy(r0.at[pl.ds(0, 40)], acc.at[pl.ds(stripe0 + j * 40, 40)])
        return 0
    lax.fori_loop(0, slen // 40, _zc, 0)
    plsc.subcore_barrier()

    def _ld_idx(g, b):
        pltpu.async_copy(src_hbm.at[pl.ds(base + g * CH, CH)], sib[b], isem[b])
        pltpu.async_copy(dst_hbm.at[pl.ds(base + g * CH, CH)], dib[b], isem[b])

    def _wait_idx(b):
        pltpu.make_async_copy(src_hbm.at[pl.ds(base, CH)], sib[b], isem[b]).wait()
        pltpu.make_async_copy(dst_hbm.at[pl.ds(base, CH)], dib[b], isem[b]).wait()

    def _gather(b, u):
        pltpu.async_copy(hs_hbm.at[sib[b]], rows[u], gsem[u])

    def _wait_gather(u):
        pltpu.make_async_copy(hs_hbm.at[sib[0]], rows[u], gsem[u]).wait()

    def _scat(b, u):
        pltpu.async_copy(rows[u], acc.at[dib[b]], ssem[u], add=True)

    def _wait_scat(u):
        pltpu.make_async_copy(rows[u], acc.at[dib[0]], ssem[u]).wait()

    for g0 in range(NI - 1):
        _ld_idx(g0, g0)
    for g0 in range(NB - 1):
        _wait_idx(g0)
        _gather(g0, g0)
    _wait_idx(3)
    _gather(3, 3)
    _ld_idx(NI - 1, NI - 1)
    _wait_gather(0)
    _scat(0, 0)

    def _step(g, v):
        u = (v + 0) % NB
        w = (v + NB - 1) % NB
        b3 = (v + NB - 1) % NI
        b7 = (v + NI - 1) % NI
        _wait_scat(w)
        _wait_idx(b3)
        _gather(b3, w)
        _ld_idx(g + NI - 1, b7)
        _wait_gather(u)
        _scat(v % NI, u)

    nfull = (nch - NI) // NI * NI

    def _body(bo, _):
        for v in range(NI):
            _step(NI * bo + 1 + v, 1 + v)
        return 0
    lax.fori_loop(0, nfull // NI, _body, 0)

    for g in range(nfull + 1, nch - NI + 1):
        _step(g, g % NI)
    for g in range(nch - NI + 1, nch - NB + 1):
        v = g % NI
        u = g % NB
        w = (g + NB - 1) % NB
        _wait_scat(w)
        _wait_idx((g + NB - 1) % NI)
        _gather((g + NB - 1) % NI, w)
        _wait_gather(u)
        _scat(v, u)
    for g in range(nch - NB + 1, nch):
        u = g % NB
        _wait_gather(u)
        _scat(g % NI, u)

    for u in range(NB):
        _wait_scat(u)

    plsc.subcore_barrier()
    pltpu.sync_copy(acc.at[pl.ds(stripe0, slen)],
                    out_hbm.at[c, pl.ds(stripe0, slen)])


def _deg_call(dst3, nch):
    return pl.kernel(
        functools.partial(_deg_body, nch),
        out_type=jax.ShapeDtypeStruct((NC, 1, NPAD), jnp.float32),
        mesh=_mesh,
        scratch_types=[
            pltpu.VMEM_SHARED((NPAD,), jnp.float32),
            pltpu.VMEM((nch, CH), jnp.int32),
            pltpu.VMEM((ZCH,), jnp.float32),
            pltpu.VMEM((CH,), jnp.float32),
        ] + [pltpu.SemaphoreType.DMA] * NB,
    )(dst3)


def _prop_call(hs, srcp, dstp, N, D, nch):
    return pl.kernel(
        functools.partial(_prop_body, N, D, nch),
        out_type=jax.ShapeDtypeStruct((NC, N, D), jnp.float32),
        mesh=_mesh,
        scratch_types=[
            pltpu.VMEM_SHARED((N, D), jnp.float32),
            tuple(pltpu.VMEM((CH,), jnp.int32) for _ in range(NI)),
            tuple(pltpu.VMEM((CH,), jnp.int32) for _ in range(NI)),
            tuple(pltpu.VMEM((CH, D), jnp.float32) for _ in range(NB)),
            tuple(pltpu.SemaphoreType.DMA for _ in range(NI)),
            tuple(pltpu.SemaphoreType.DMA for _ in range(NB)),
            tuple(pltpu.SemaphoreType.DMA for _ in range(NB)),
        ],
    )(hs, srcp, dstp)


BLK = 2000


def _mm_scale_body(x_ref, w_ref, deg_ref, hs_ref):
    dinv = lax.rsqrt(deg_ref[:, 0:1] + deg_ref[:, 1:2] + 1.0)
    hs_ref[...] = jnp.dot(x_ref[...], w_ref[...],
                          preferred_element_type=jnp.float32) * dinv


def _mid_body(a_ref, hs_ref, deg_ref, b_ref, w_ref, out_ref):
    dinv = lax.rsqrt(deg_ref[:, 0:1] + deg_ref[:, 1:2] + 1.0)
    h = (a_ref[0] + a_ref[1] + hs_ref[...]) * dinv + b_ref[...]
    h = jnp.maximum(h, 0.0)
    out_ref[...] = jnp.dot(h, w_ref[...],
                           preferred_element_type=jnp.float32) * dinv


def _final_body(a_ref, hs_ref, deg_ref, b_ref, out_ref):
    dinv = lax.rsqrt(deg_ref[:, 0:1] + deg_ref[:, 1:2] + 1.0)
    out_ref[...] = (a_ref[0] + a_ref[1] + hs_ref[...]) * dinv + b_ref[...]


def _row_spec(D):
    return pl.BlockSpec((BLK, D), lambda i: (i, 0))


def _mm_scale(x, W, degT, N, D):
    return pl.pallas_call(
        _mm_scale_body,
        grid=(N // BLK,),
        in_specs=[
            _row_spec(D),
            pl.BlockSpec((D, D), lambda i: (0, 0)),
            pl.BlockSpec((BLK, 2), lambda i: (i, 0)),
        ],
        out_specs=_row_spec(D),
        out_shape=jax.ShapeDtypeStruct((N, D), jnp.float32),
    )(x, W, degT)


def _acc_spec(D):
    return pl.BlockSpec((NC, BLK, D), lambda i: (0, i, 0))


def _mid(acc, hs, degT, b, W, N, D):
    return pl.pallas_call(
        _mid_body,
        grid=(N // BLK,),
        in_specs=[
            _acc_spec(D), _row_spec(D),
            pl.BlockSpec((BLK, 2), lambda i: (i, 0)),
            pl.BlockSpec((1, D), lambda i: (0, 0)),
            pl.BlockSpec((D, D), lambda i: (0, 0)),
        ],
        out_specs=_row_spec(D),
        out_shape=jax.ShapeDtypeStruct((N, D), jnp.float32),
    )(acc, hs, degT, b, W)


def _final(acc, hs, degT, b, N, D):
    return pl.pallas_call(
        _final_body,
        grid=(N // BLK,),
        in_specs=[
            _acc_spec(D), _row_spec(D),
            pl.BlockSpec((BLK, 2), lambda i: (i, 0)),
            pl.BlockSpec((1, D), lambda i: (0, 0)),
        ],
        out_specs=_row_spec(D),
        out_shape=jax.ShapeDtypeStruct((N, D), jnp.float32),
    )(acc, hs, degT, b)


def kernel(x, edge_index, cache_name, W1, b1, W2, b2):
    N, D = x.shape
    E = edge_index.shape[1]
    nch = EPT // CH
    dstp = lax.optimization_barrier(edge_index[1])
    srcp = lax.optimization_barrier(edge_index[0])
    dst3 = dstp.reshape(NW, nch, CH)

    deg_parts = _deg_call(dst3, nch)
    degT = deg_parts[:, 0, :].T[:N]

    hs1 = _mm_scale(x, W1, degT, N, D)
    acc1 = _prop_call(hs1, srcp, dstp, N, D, nch)
    hs2 = _mid(acc1, hs1, degT, b1.reshape(1, D), W2, N, D)
    acc2 = _prop_call(hs2, srcp, dstp, N, D, nch)
    return _final(acc2, hs2, degT, b2.reshape(1, D), N, D)
